# baseline (speedup 1.0000x reference)
import functools

import jax
import jax.numpy as jnp
from jax import lax
from jax.experimental import pallas as pl
from jax.experimental.pallas import tpu as pltpu
from jax.experimental.pallas import tpu_sc as plsc

N_NODES = 10000
D = 128
E_TOTAL = 320000

NC = 2
NS = 16
NW = NC * NS
E_W = E_TOTAL // NW
CHUNK = 40
NCHUNK = E_W // CHUNK
N_PAD = 10240
ROWS_W = N_PAD // NS


NSLOT = 5
LA = 3
GA = 2


def _sc_agg_body(with_counts, x_hbm, src_hbm, dst_hbm, zrows_hbm, zcnt_hbm,
                 ones_hbm, out_hbm, cntout_hbm, srcc, dstc, rows, ones,
                 acc, cnt, sem_r, sem_d, sem_g, sem_s, sem_c, sem_z):
  c = lax.axis_index("c")
  s = lax.axis_index("s")
  wid = s * NC + c
  e0 = wid * E_W
  row0 = s * ROWS_W

  pltpu.async_copy(zrows_hbm, acc.at[pl.ds(row0, ROWS_W)], sem_z)
  if with_counts:
    pltpu.sync_copy(ones_hbm, ones)
    pltpu.sync_copy(zcnt_hbm, cnt.at[pl.ds(row0, ROWS_W)])

  def d_src(p, i):
    return pltpu.make_async_copy(src_hbm.at[pl.ds(e0 + i * CHUNK, CHUNK)],
                                 srcc[p], sem_r[p])

  def d_dst(p, i):
    return pltpu.make_async_copy(dst_hbm.at[pl.ds(e0 + i * CHUNK, CHUNK)],
                                 dstc[p], sem_d[p])

  def d_gat(p):
    return pltpu.make_async_copy(x_hbm.at[srcc[p]], rows[p], sem_g[p])

  def d_sca(p):
    return pltpu.make_async_copy(rows[p], acc.at[dstc[p]], sem_s[p])

  def d_cnt(p):
    return pltpu.make_async_copy(ones, cnt.at[dstc[p]], sem_c[p])

  def load(p, i):
    d_src(p, i).start()
    d_dst(p, i).start()

  def gather(p, i):
    d_src(p, i).wait()
    d_gat(p).start()

  def scatter(p, i):
    d_dst(p, i).wait()
    d_gat(p).wait()
    d_sca(p).start(add=True)
    if with_counts:
      d_cnt(p).start(add=True)

  def drain(p):
    d_sca(p).wait()
    if with_counts:
      d_cnt(p).wait()

  for j in range(LA):
    load(j % NSLOT, j)
  for j in range(GA):
    gather(j % NSLOT, j)
  pltpu.make_async_copy(zrows_hbm, acc.at[pl.ds(row0, ROWS_W)], sem_z).wait()
  plsc.subcore_barrier()

  def pipe_body(k, carry):
    base = k * NSLOT
    for p in range(NSLOT):
      i = base + p
      ql = (p + LA) % NSLOT
      qg = (p + GA) % NSLOT

      @pl.when(i + LA < NCHUNK)
      def _():
        @pl.when(i + LA >= NSLOT)
        def _():
          drain(ql)
        load(ql, i + LA)

      @pl.when(i + GA < NCHUNK)
      def _():
        gather(qg, i + GA)

      @pl.when(i < NCHUNK)
      def _():
        scatter(p, i)
    return carry

  lax.fori_loop(0, (NCHUNK + NSLOT - 1) // NSLOT, pipe_body, 0)
  for p in range(NSLOT):
    drain(p)
  plsc.subcore_barrier()

  pltpu.sync_copy(acc.at[pl.ds(row0, ROWS_W)],
                  out_hbm.at[c, pl.ds(row0, ROWS_W)])
  if with_counts:
    pltpu.sync_copy(cnt.at[pl.ds(row0, ROWS_W)],
                    cntout_hbm.at[c, pl.ds(row0, ROWS_W)])


def _make_sc_agg(with_counts):
  mesh = plsc.VectorSubcoreMesh(core_axis_name="c", subcore_axis_name="s")
  out_type = [jax.ShapeDtypeStruct((NC, N_PAD, D), jnp.float32)]
  if with_counts:
    out_type.append(jax.ShapeDtypeStruct((NC, N_PAD), jnp.float32))
  scratch_types = [
      [pltpu.VMEM((CHUNK,), jnp.int32) for _ in range(NSLOT)],
      [pltpu.VMEM((CHUNK,), jnp.int32) for _ in range(NSLOT)],
      [pltpu.VMEM((CHUNK, D), jnp.float32) for _ in range(NSLOT)],
      pltpu.VMEM((CHUNK,), jnp.float32),
      pltpu.VMEM_SHARED((N_PAD, D), jnp.float32),
      pltpu.VMEM_SHARED((N_PAD,), jnp.float32),
      [pltpu.SemaphoreType.DMA for _ in range(NSLOT)],
      [pltpu.SemaphoreType.DMA for _ in range(NSLOT)],
      [pltpu.SemaphoreType.DMA for _ in range(NSLOT)],
      [pltpu.SemaphoreType.DMA for _ in range(NSLOT)],
      [pltpu.SemaphoreType.DMA for _ in range(NSLOT)],
      pltpu.SemaphoreType.DMA,
  ]

  if with_counts:
    def body(x_hbm, src_hbm, dst_hbm, zrows_hbm, zcnt_hbm, ones_hbm,
             out_hbm, cntout_hbm, srcc, dstc, rows, ones, acc, cnt,
             sem_r, sem_d, sem_g, sem_s, sem_c, sem_z):
      _sc_agg_body(True, x_hbm, src_hbm, dst_hbm, zrows_hbm, zcnt_hbm,
                   ones_hbm, out_hbm, cntout_hbm, srcc, dstc, rows, ones,
                   acc, cnt, sem_r, sem_d, sem_g, sem_s, sem_c, sem_z)
  else:
    def body(x_hbm, src_hbm, dst_hbm, zrows_hbm, zcnt_hbm, ones_hbm,
             out_hbm, srcc, dstc, rows, ones, acc, cnt,
             sem_r, sem_d, sem_g, sem_s, sem_c, sem_z):
      _sc_agg_body(False, x_hbm, src_hbm, dst_hbm, zrows_hbm, zcnt_hbm,
                   ones_hbm, out_hbm, None, srcc, dstc, rows, ones,
                   acc, cnt, sem_r, sem_d, sem_g, sem_s, sem_c, sem_z)

  return pl.kernel(body, out_type=out_type, mesh=mesh,
                   scratch_types=scratch_types)


_sc_agg_with_counts = _make_sc_agg(True)
_sc_agg_no_counts = _make_sc_agg(False)

BR = 400


def _tc_layer_body(p_ref, c_ref, x_ref, wl_ref, wr_ref, b_ref, o_ref):
  cnt = jnp.maximum(c_ref[0] + c_ref[1], 1.0)
  agg = (p_ref[0] + p_ref[1]) / cnt
  y = lax.dot_general(agg, wl_ref[...], (((1,), (1,)), ((), ())),
                      preferred_element_type=jnp.float32)
  y = y + lax.dot_general(x_ref[...], wr_ref[...], (((1,), (1,)), ((), ())),
                          preferred_element_type=jnp.float32)
  o_ref[...] = jnp.maximum(y + b_ref[...], 0.0)


def _tc_layer(P, C, x, Wl, Wr, b):
  grid = (N_NODES // BR,)
  return pl.pallas_call(
      _tc_layer_body,
      grid=grid,
      in_specs=[
          pl.BlockSpec((NC, BR, D), lambda i: (0, i, 0)),
          pl.BlockSpec((NC, BR, 1), lambda i: (0, i, 0)),
          pl.BlockSpec((BR, D), lambda i: (i, 0)),
          pl.BlockSpec((D, D), lambda i: (0, 0)),
          pl.BlockSpec((D, D), lambda i: (0, 0)),
          pl.BlockSpec((1, D), lambda i: (0, 0)),
      ],
      out_specs=pl.BlockSpec((BR, D), lambda i: (i, 0)),
      out_shape=jax.ShapeDtypeStruct((N_NODES, D), jnp.float32),
  )(P, C.reshape(NC, N_PAD, 1), x, Wl, Wr, b.reshape(1, D))


def _tc_final_body(p_ref, c_ref, h_ref, wl_ref, wr_ref, b_ref, wc_ref,
                   bc_ref, o_ref):
  cnt = jnp.maximum(c_ref[0] + c_ref[1], 1.0)
  agg = (p_ref[0] + p_ref[1]) / cnt
  y = lax.dot_general(agg, wl_ref[...], (((1,), (1,)), ((), ())),
                      preferred_element_type=jnp.float32)
  y = y + lax.dot_general(h_ref[...], wr_ref[...], (((1,), (1,)), ((), ())),
                          preferred_element_type=jnp.float32)
  h2 = jnp.maximum(y + b_ref[...], 0.0)
  logits = jnp.sum(h2 * wc_ref[...], axis=1, keepdims=True)
  o_ref[...] = jax.nn.sigmoid(logits + bc_ref[0])


def _tc_final(P, C, h, Wl, Wr, b, Wc, bc):
  grid = (N_NODES // BR,)
  return pl.pallas_call(
      _tc_final_body,
      grid=grid,
      in_specs=[
          pl.BlockSpec((NC, BR, D), lambda i: (0, i, 0)),
          pl.BlockSpec((NC, BR, 1), lambda i: (0, i, 0)),
          pl.BlockSpec((BR, D), lambda i: (i, 0)),
          pl.BlockSpec((D, D), lambda i: (0, 0)),
          pl.BlockSpec((D, D), lambda i: (0, 0)),
          pl.BlockSpec((1, D), lambda i: (0, 0)),
          pl.BlockSpec((1, D), lambda i: (0, 0)),
          pl.BlockSpec(memory_space=pltpu.SMEM),
      ],
      out_specs=pl.BlockSpec((BR, 1), lambda i: (i, 0)),
      out_shape=jax.ShapeDtypeStruct((N_NODES, 1), jnp.float32),
  )(P, C.reshape(NC, N_PAD, 1), h, Wl, Wr, b.reshape(1, D), Wc.reshape(1, D),
    bc)


def kernel(x, edge_index, W1l, b1, W1r, W2l, b2, W2r, Wc, bc):
  src = edge_index[0]
  dst = edge_index[1]
  zrows = jnp.zeros((ROWS_W, D), jnp.float32)
  zcnt = jnp.zeros((ROWS_W,), jnp.float32)
  ones = jnp.ones((CHUNK,), jnp.float32)

  P1, C1 = _sc_agg_with_counts(x, src, dst, zrows, zcnt, ones)
  h = _tc_layer(P1, C1, x, W1l, W1r, b1)
  (P2,) = _sc_agg_no_counts(h, src, dst, zrows, zcnt, ones)
  out = _tc_final(P2, C1, h, W2l, W2r, b2, Wc, bc)
  return out


# speedup vs baseline: 1.0749x; 1.0749x over previous
import functools

import jax
import jax.numpy as jnp
from jax import lax
from jax.experimental import pallas as pl
from jax.experimental.pallas import tpu as pltpu
from jax.experimental.pallas import tpu_sc as plsc

N_NODES = 10000
D = 128
E_TOTAL = 320000

NC = 2
NS = 16
NW = NC * NS
E_W = E_TOTAL // NW
CHUNK = 80
NCHUNK = E_W // CHUNK
N_PAD = 10240
ROWS_W = N_PAD // NS


NSLOT = 4
LA = 2
GA = 1


def _sc_agg_body(with_counts, x_hbm, src_hbm, dst_hbm, zrows_hbm, zcnt_hbm,
                 ones_hbm, out_hbm, cntout_hbm, srcc, dstc, rows, ones,
                 acc, cnt, sem_r, sem_d, sem_g, sem_s, sem_c, sem_z):
  c = lax.axis_index("c")
  s = lax.axis_index("s")
  wid = s * NC + c
  e0 = wid * E_W
  row0 = s * ROWS_W

  pltpu.async_copy(zrows_hbm, acc.at[pl.ds(row0, ROWS_W)], sem_z)
  if with_counts:
    pltpu.sync_copy(ones_hbm, ones)
    pltpu.sync_copy(zcnt_hbm, cnt.at[pl.ds(row0, ROWS_W)])

  def d_src(p, i):
    return pltpu.make_async_copy(src_hbm.at[pl.ds(e0 + i * CHUNK, CHUNK)],
                                 srcc[p], sem_r[p])

  def d_dst(p, i):
    return pltpu.make_async_copy(dst_hbm.at[pl.ds(e0 + i * CHUNK, CHUNK)],
                                 dstc[p], sem_d[p])

  def d_gat(p):
    return pltpu.make_async_copy(x_hbm.at[srcc[p]], rows[p], sem_g[p])

  def d_sca(p):
    return pltpu.make_async_copy(rows[p], acc.at[dstc[p]], sem_s[p])

  def d_cnt(p):
    return pltpu.make_async_copy(ones, cnt.at[dstc[p]], sem_c[p])

  def load(p, i):
    d_src(p, i).start()
    d_dst(p, i).start()

  def gather(p, i):
    d_src(p, i).wait()
    d_gat(p).start()

  def scatter(p, i):
    d_dst(p, i).wait()
    d_gat(p).wait()
    d_sca(p).start(add=True)
    if with_counts:
      d_cnt(p).start(add=True)

  def drain(p):
    d_sca(p).wait()
    if with_counts:
      d_cnt(p).wait()

  for j in range(LA):
    load(j % NSLOT, j)
  for j in range(GA):
    gather(j % NSLOT, j)
  pltpu.make_async_copy(zrows_hbm, acc.at[pl.ds(row0, ROWS_W)], sem_z).wait()
  plsc.subcore_barrier()

  def pipe_body(k, carry):
    base = k * NSLOT
    for p in range(NSLOT):
      i = base + p
      ql = (p + LA) % NSLOT
      qg = (p + GA) % NSLOT

      @pl.when(i + LA < NCHUNK)
      def _():
        @pl.when(i + LA >= NSLOT)
        def _():
          drain(ql)
        load(ql, i + LA)

      @pl.when(i + GA < NCHUNK)
      def _():
        gather(qg, i + GA)

      @pl.when(i < NCHUNK)
      def _():
        scatter(p, i)
    return carry

  lax.fori_loop(0, (NCHUNK + NSLOT - 1) // NSLOT, pipe_body, 0)
  for p in range(NSLOT):
    drain(p)
  plsc.subcore_barrier()

  pltpu.sync_copy(acc.at[pl.ds(row0, ROWS_W)],
                  out_hbm.at[c, pl.ds(row0, ROWS_W)])
  if with_counts:
    pltpu.sync_copy(cnt.at[pl.ds(row0, ROWS_W)],
                    cntout_hbm.at[c, pl.ds(row0, ROWS_W)])


def _make_sc_agg(with_counts):
  mesh = plsc.VectorSubcoreMesh(core_axis_name="c", subcore_axis_name="s")
  out_type = [jax.ShapeDtypeStruct((NC, N_PAD, D), jnp.float32)]
  if with_counts:
    out_type.append(jax.ShapeDtypeStruct((NC, N_PAD), jnp.float32))
  scratch_types = [
      [pltpu.VMEM((CHUNK,), jnp.int32) for _ in range(NSLOT)],
      [pltpu.VMEM((CHUNK,), jnp.int32) for _ in range(NSLOT)],
      [pltpu.VMEM((CHUNK, D), jnp.float32) for _ in range(NSLOT)],
      pltpu.VMEM((CHUNK,), jnp.float32),
      pltpu.VMEM_SHARED((N_PAD, D), jnp.float32),
      pltpu.VMEM_SHARED((N_PAD,), jnp.float32),
      [pltpu.SemaphoreType.DMA for _ in range(NSLOT)],
      [pltpu.SemaphoreType.DMA for _ in range(NSLOT)],
      [pltpu.SemaphoreType.DMA for _ in range(NSLOT)],
      [pltpu.SemaphoreType.DMA for _ in range(NSLOT)],
      [pltpu.SemaphoreType.DMA for _ in range(NSLOT)],
      pltpu.SemaphoreType.DMA,
  ]

  if with_counts:
    def body(x_hbm, src_hbm, dst_hbm, zrows_hbm, zcnt_hbm, ones_hbm,
             out_hbm, cntout_hbm, srcc, dstc, rows, ones, acc, cnt,
             sem_r, sem_d, sem_g, sem_s, sem_c, sem_z):
      _sc_agg_body(True, x_hbm, src_hbm, dst_hbm, zrows_hbm, zcnt_hbm,
                   ones_hbm, out_hbm, cntout_hbm, srcc, dstc, rows, ones,
                   acc, cnt, sem_r, sem_d, sem_g, sem_s, sem_c, sem_z)
  else:
    def body(x_hbm, src_hbm, dst_hbm, zrows_hbm, zcnt_hbm, ones_hbm,
             out_hbm, srcc, dstc, rows, ones, acc, cnt,
             sem_r, sem_d, sem_g, sem_s, sem_c, sem_z):
      _sc_agg_body(False, x_hbm, src_hbm, dst_hbm, zrows_hbm, zcnt_hbm,
                   ones_hbm, out_hbm, None, srcc, dstc, rows, ones,
                   acc, cnt, sem_r, sem_d, sem_g, sem_s, sem_c, sem_z)

  return pl.kernel(body, out_type=out_type, mesh=mesh,
                   scratch_types=scratch_types)


_sc_agg_with_counts = _make_sc_agg(True)
_sc_agg_no_counts = _make_sc_agg(False)

BR = 400


def _tc_layer_body(p_ref, c_ref, x_ref, wl_ref, wr_ref, b_ref, o_ref):
  cnt = jnp.maximum(c_ref[0] + c_ref[1], 1.0)
  agg = (p_ref[0] + p_ref[1]) / cnt
  y = lax.dot_general(agg, wl_ref[...], (((1,), (1,)), ((), ())),
                      preferred_element_type=jnp.float32)
  y = y + lax.dot_general(x_ref[...], wr_ref[...], (((1,), (1,)), ((), ())),
                          preferred_element_type=jnp.float32)
  o_ref[...] = jnp.maximum(y + b_ref[...], 0.0)


def _tc_layer(P, C, x, Wl, Wr, b):
  grid = (N_NODES // BR,)
  return pl.pallas_call(
      _tc_layer_body,
      grid=grid,
      in_specs=[
          pl.BlockSpec((NC, BR, D), lambda i: (0, i, 0)),
          pl.BlockSpec((NC, BR, 1), lambda i: (0, i, 0)),
          pl.BlockSpec((BR, D), lambda i: (i, 0)),
          pl.BlockSpec((D, D), lambda i: (0, 0)),
          pl.BlockSpec((D, D), lambda i: (0, 0)),
          pl.BlockSpec((1, D), lambda i: (0, 0)),
      ],
      out_specs=pl.BlockSpec((BR, D), lambda i: (i, 0)),
      out_shape=jax.ShapeDtypeStruct((N_NODES, D), jnp.float32),
  )(P, C.reshape(NC, N_PAD, 1), x, Wl, Wr, b.reshape(1, D))


def _tc_final_body(p_ref, c_ref, h_ref, wl_ref, wr_ref, b_ref, wc_ref,
                   bc_ref, o_ref):
  cnt = jnp.maximum(c_ref[0] + c_ref[1], 1.0)
  agg = (p_ref[0] + p_ref[1]) / cnt
  y = lax.dot_general(agg, wl_ref[...], (((1,), (1,)), ((), ())),
                      preferred_element_type=jnp.float32)
  y = y + lax.dot_general(h_ref[...], wr_ref[...], (((1,), (1,)), ((), ())),
                          preferred_element_type=jnp.float32)
  h2 = jnp.maximum(y + b_ref[...], 0.0)
  logits = jnp.sum(h2 * wc_ref[...], axis=1, keepdims=True)
  o_ref[...] = jax.nn.sigmoid(logits + bc_ref[0])


def _tc_final(P, C, h, Wl, Wr, b, Wc, bc):
  grid = (N_NODES // BR,)
  return pl.pallas_call(
      _tc_final_body,
      grid=grid,
      in_specs=[
          pl.BlockSpec((NC, BR, D), lambda i: (0, i, 0)),
          pl.BlockSpec((NC, BR, 1), lambda i: (0, i, 0)),
          pl.BlockSpec((BR, D), lambda i: (i, 0)),
          pl.BlockSpec((D, D), lambda i: (0, 0)),
          pl.BlockSpec((D, D), lambda i: (0, 0)),
          pl.BlockSpec((1, D), lambda i: (0, 0)),
          pl.BlockSpec((1, D), lambda i: (0, 0)),
          pl.BlockSpec(memory_space=pltpu.SMEM),
      ],
      out_specs=pl.BlockSpec((BR, 1), lambda i: (i, 0)),
      out_shape=jax.ShapeDtypeStruct((N_NODES, 1), jnp.float32),
  )(P, C.reshape(NC, N_PAD, 1), h, Wl, Wr, b.reshape(1, D), Wc.reshape(1, D),
    bc)


def kernel(x, edge_index, W1l, b1, W1r, W2l, b2, W2r, Wc, bc):
  src = edge_index[0]
  dst = edge_index[1]
  zrows = jnp.zeros((ROWS_W, D), jnp.float32)
  zcnt = jnp.zeros((ROWS_W,), jnp.float32)
  ones = jnp.ones((CHUNK,), jnp.float32)

  P1, C1 = _sc_agg_with_counts(x, src, dst, zrows, zcnt, ones)
  h = _tc_layer(P1, C1, x, W1l, W1r, b1)
  (P2,) = _sc_agg_no_counts(h, src, dst, zrows, zcnt, ones)
  out = _tc_final(P2, C1, h, W2l, W2r, b2, Wc, bc)
  return out


# speedup vs baseline: 1.1707x; 1.0891x over previous
import functools

import jax
import jax.numpy as jnp
from jax import lax
from jax.experimental import pallas as pl
from jax.experimental.pallas import tpu as pltpu
from jax.experimental.pallas import tpu_sc as plsc

N_NODES = 10000
D = 128
E_TOTAL = 320000

NC = 2
NS = 16
NW = NC * NS
E_W = E_TOTAL // NW
CHUNK = 80
NCHUNK = E_W // CHUNK
N_PAD = 10240
ROWS_W = N_PAD // NS


NSLOT = 4
LA = 2
GA = 1


def _sc_agg_body(with_counts, x_hbm, src_hbm, dst_hbm, zrows_hbm, zcnt_hbm,
                 ones_hbm, out_hbm, cntout_hbm, srcc, dstc, rows, ones,
                 acc, cnt, sem_r, sem_d, sem_g, sem_s, sem_c, sem_z):
  c = lax.axis_index("c")
  s = lax.axis_index("s")
  wid = s * NC + c
  e0 = wid * E_W
  row0 = s * ROWS_W

  pltpu.async_copy(zrows_hbm, acc.at[pl.ds(row0, ROWS_W)], sem_z)
  if with_counts:
    pltpu.sync_copy(ones_hbm, ones)
    pltpu.sync_copy(zcnt_hbm, cnt.at[pl.ds(row0, ROWS_W)])

  def d_src(p, i):
    return pltpu.make_async_copy(src_hbm.at[pl.ds(e0 + i * CHUNK, CHUNK)],
                                 srcc[p], sem_r[p])

  def d_dst(p, i):
    return pltpu.make_async_copy(dst_hbm.at[pl.ds(e0 + i * CHUNK, CHUNK)],
                                 dstc[p], sem_d[p])

  def d_gat(p):
    return pltpu.make_async_copy(x_hbm.at[srcc[p]], rows[p], sem_g[p])

  def d_sca(p):
    return pltpu.make_async_copy(rows[p], acc.at[dstc[p]], sem_s[p])

  def d_cnt(p):
    return pltpu.make_async_copy(ones, cnt.at[dstc[p]], sem_c[p])

  def load(p, i):
    d_src(p, i).start()
    d_dst(p, i).start()

  def gather(p, i):
    d_src(p, i).wait()
    d_gat(p).start()

  def scatter(p, i):
    d_dst(p, i).wait()
    d_gat(p).wait()
    d_sca(p).start(add=True)
    if with_counts:
      d_cnt(p).start(add=True)

  def drain(p):
    d_sca(p).wait()
    if with_counts:
      d_cnt(p).wait()

  for j in range(LA):
    load(j % NSLOT, j)
  for j in range(GA):
    gather(j % NSLOT, j)
  pltpu.make_async_copy(zrows_hbm, acc.at[pl.ds(row0, ROWS_W)], sem_z).wait()
  plsc.subcore_barrier()

  def pipe_body(k, carry):
    base = k * NSLOT
    for p in range(NSLOT):
      i = base + p
      ql = (p + LA) % NSLOT
      qg = (p + GA) % NSLOT

      @pl.when(i + LA < NCHUNK)
      def _():
        @pl.when(i + LA >= NSLOT)
        def _():
          drain(ql)
        load(ql, i + LA)

      @pl.when(i + GA < NCHUNK)
      def _():
        gather(qg, i + GA)

      @pl.when(i < NCHUNK)
      def _():
        scatter(p, i)
    return carry

  lax.fori_loop(0, (NCHUNK + NSLOT - 1) // NSLOT, pipe_body, 0)
  for p in range(NSLOT):
    drain(p)
  plsc.subcore_barrier()

  pltpu.sync_copy(acc.at[pl.ds(row0, ROWS_W)],
                  out_hbm.at[c, pl.ds(row0, ROWS_W)])
  if with_counts:
    pltpu.sync_copy(cnt.at[pl.ds(row0, ROWS_W)],
                    cntout_hbm.at[c, pl.ds(row0, ROWS_W)])


def _make_sc_agg(with_counts):
  mesh = plsc.VectorSubcoreMesh(core_axis_name="c", subcore_axis_name="s")
  out_type = [jax.ShapeDtypeStruct((NC, N_PAD, D), jnp.float32)]
  if with_counts:
    out_type.append(jax.ShapeDtypeStruct((NC, N_PAD), jnp.float32))
  scratch_types = [
      [pltpu.VMEM((CHUNK,), jnp.int32) for _ in range(NSLOT)],
      [pltpu.VMEM((CHUNK,), jnp.int32) for _ in range(NSLOT)],
      [pltpu.VMEM((CHUNK, D), jnp.float32) for _ in range(NSLOT)],
      pltpu.VMEM((CHUNK,), jnp.float32),
      pltpu.VMEM_SHARED((N_PAD, D), jnp.float32),
      pltpu.VMEM_SHARED((N_PAD,), jnp.float32),
      [pltpu.SemaphoreType.DMA for _ in range(NSLOT)],
      [pltpu.SemaphoreType.DMA for _ in range(NSLOT)],
      [pltpu.SemaphoreType.DMA for _ in range(NSLOT)],
      [pltpu.SemaphoreType.DMA for _ in range(NSLOT)],
      [pltpu.SemaphoreType.DMA for _ in range(NSLOT)],
      pltpu.SemaphoreType.DMA,
  ]

  if with_counts:
    def body(x_hbm, src_hbm, dst_hbm, zrows_hbm, zcnt_hbm, ones_hbm,
             out_hbm, cntout_hbm, srcc, dstc, rows, ones, acc, cnt,
             sem_r, sem_d, sem_g, sem_s, sem_c, sem_z):
      _sc_agg_body(True, x_hbm, src_hbm, dst_hbm, zrows_hbm, zcnt_hbm,
                   ones_hbm, out_hbm, cntout_hbm, srcc, dstc, rows, ones,
                   acc, cnt, sem_r, sem_d, sem_g, sem_s, sem_c, sem_z)
  else:
    def body(x_hbm, src_hbm, dst_hbm, zrows_hbm, zcnt_hbm, ones_hbm,
             out_hbm, srcc, dstc, rows, ones, acc, cnt,
             sem_r, sem_d, sem_g, sem_s, sem_c, sem_z):
      _sc_agg_body(False, x_hbm, src_hbm, dst_hbm, zrows_hbm, zcnt_hbm,
                   ones_hbm, out_hbm, None, srcc, dstc, rows, ones,
                   acc, cnt, sem_r, sem_d, sem_g, sem_s, sem_c, sem_z)

  return pl.kernel(body, out_type=out_type, mesh=mesh,
                   scratch_types=scratch_types,
                   compiler_params=pltpu.CompilerParams(
                       use_tc_tiling_on_sc=True))


_sc_agg_with_counts = _make_sc_agg(True)
_sc_agg_no_counts = _make_sc_agg(False)

BR = 2000


def _tc_layer_body(p_ref, c_ref, x_ref, wl_ref, wr_ref, b_ref, o_ref):
  cnt = jnp.maximum(c_ref[0] + c_ref[1], 1.0)
  agg = (p_ref[0] + p_ref[1]) / cnt
  y = lax.dot_general(agg, wl_ref[...], (((1,), (1,)), ((), ())),
                      preferred_element_type=jnp.float32)
  y = y + lax.dot_general(x_ref[...], wr_ref[...], (((1,), (1,)), ((), ())),
                          preferred_element_type=jnp.float32)
  o_ref[...] = jnp.maximum(y + b_ref[...], 0.0)


def _tc_layer(P, C, x, Wl, Wr, b):
  grid = (N_NODES // BR,)
  return pl.pallas_call(
      _tc_layer_body,
      grid=grid,
      in_specs=[
          pl.BlockSpec((NC, BR, D), lambda i: (0, i, 0)),
          pl.BlockSpec((NC, BR, 1), lambda i: (0, i, 0)),
          pl.BlockSpec((BR, D), lambda i: (i, 0)),
          pl.BlockSpec((D, D), lambda i: (0, 0)),
          pl.BlockSpec((D, D), lambda i: (0, 0)),
          pl.BlockSpec((1, D), lambda i: (0, 0)),
      ],
      out_specs=pl.BlockSpec((BR, D), lambda i: (i, 0)),
      out_shape=jax.ShapeDtypeStruct((N_NODES, D), jnp.float32),
  )(P, C.reshape(NC, N_PAD, 1), x, Wl, Wr, b.reshape(1, D))


def _tc_final_body(p_ref, c_ref, h_ref, wl_ref, wr_ref, b_ref, wc_ref,
                   bc_ref, o_ref):
  cnt = jnp.maximum(c_ref[0] + c_ref[1], 1.0)
  agg = (p_ref[0] + p_ref[1]) / cnt
  y = lax.dot_general(agg, wl_ref[...], (((1,), (1,)), ((), ())),
                      preferred_element_type=jnp.float32)
  y = y + lax.dot_general(h_ref[...], wr_ref[...], (((1,), (1,)), ((), ())),
                          preferred_element_type=jnp.float32)
  h2 = jnp.maximum(y + b_ref[...], 0.0)
  logits = jnp.sum(h2 * wc_ref[...], axis=1, keepdims=True)
  o_ref[...] = jax.nn.sigmoid(logits + bc_ref[0])


def _tc_final(P, C, h, Wl, Wr, b, Wc, bc):
  grid = (N_NODES // BR,)
  return pl.pallas_call(
      _tc_final_body,
      grid=grid,
      in_specs=[
          pl.BlockSpec((NC, BR, D), lambda i: (0, i, 0)),
          pl.BlockSpec((NC, BR, 1), lambda i: (0, i, 0)),
          pl.BlockSpec((BR, D), lambda i: (i, 0)),
          pl.BlockSpec((D, D), lambda i: (0, 0)),
          pl.BlockSpec((D, D), lambda i: (0, 0)),
          pl.BlockSpec((1, D), lambda i: (0, 0)),
          pl.BlockSpec((1, D), lambda i: (0, 0)),
          pl.BlockSpec(memory_space=pltpu.SMEM),
      ],
      out_specs=pl.BlockSpec((BR, 1), lambda i: (i, 0)),
      out_shape=jax.ShapeDtypeStruct((N_NODES, 1), jnp.float32),
  )(P, C.reshape(NC, N_PAD, 1), h, Wl, Wr, b.reshape(1, D), Wc.reshape(1, D),
    bc)


def kernel(x, edge_index, W1l, b1, W1r, W2l, b2, W2r, Wc, bc):
  src = edge_index[0]
  dst = edge_index[1]
  zrows = jnp.zeros((ROWS_W, D), jnp.float32)
  zcnt = jnp.zeros((ROWS_W,), jnp.float32)
  ones = jnp.ones((CHUNK,), jnp.float32)

  P1, C1 = _sc_agg_with_counts(x, src, dst, zrows, zcnt, ones)
  h = _tc_layer(P1, C1, x, W1l, W1r, b1)
  (P2,) = _sc_agg_no_counts(h, src, dst, zrows, zcnt, ones)
  out = _tc_final(P2, C1, h, W2l, W2r, b2, Wc, bc)
  return out


# speedup vs baseline: 1.1999x; 1.0250x over previous
import jax
import jax.numpy as jnp
from jax import lax
from jax.experimental import pallas as pl
from jax.experimental.pallas import tpu as pltpu
from jax.experimental.pallas import tpu_sc as plsc

N_NODES = 10000
D = 128
E_TOTAL = 320000

NC = 2
NS = 16
NW = NC * NS
CHUNK = 128
E_W = 10240
E_PAD = E_W * NW
NCHUNK = E_W // CHUNK
N_PAD = 10016
N_CNT = 10112
ROWS_W = 624
ROWS_LAST = N_PAD - (NS - 1) * ROWS_W
NROW = 3
NIDX = 4


def _sc_agg_body(with_counts, x_hbm, ei_hbm, zrows_hbm, zcnt_hbm,
                 ones_hbm, out_hbm, cntout_hbm, idxb, rows, ones,
                 acc, cnt, sem_r, sem_g, sem_s, sem_c, sem_z):
  c = lax.axis_index("c")
  s = lax.axis_index("s")
  wid = s * NC + c
  e0 = wid * E_W
  row0 = s * ROWS_W

  last = s == NS - 1

  @pl.when(last)
  def _():
    pltpu.async_copy(zrows_hbm, acc.at[pl.ds(row0, ROWS_LAST)], sem_z)

  @pl.when(jnp.logical_not(last))
  def _():
    pltpu.async_copy(zrows_hbm.at[pl.ds(0, ROWS_W)],
                     acc.at[pl.ds(row0, ROWS_W)], sem_z)

  if with_counts:
    pltpu.sync_copy(ones_hbm, ones)

    @pl.when(s == 0)
    def _():
      pltpu.sync_copy(zcnt_hbm, cnt)

  def d_idx(q, i):
    return pltpu.make_async_copy(
        ei_hbm.at[:, pl.ds(e0 + i * CHUNK, CHUNK)], idxb[q].at[:, 0],
        sem_r[q])

  def d_gat(p, q):
    return pltpu.make_async_copy(x_hbm.at[idxb[q].at[0, 0]], rows[p],
                                 sem_g[p])

  def d_sca(p, q):
    return pltpu.make_async_copy(rows[p], acc.at[idxb[q].at[1, 0]], sem_s[p])

  def d_cnt(q):
    return pltpu.make_async_copy(ones, cnt.at[idxb[q].at[1, 0]], sem_c[q])

  def load(q, i):
    d_idx(q, i).start()

  def gather(p, q, i):
    d_idx(q, i).wait()
    d_gat(p, q).start()

  def scatter(p, q):
    d_gat(p, q).wait()
    d_sca(p, q).start(add=True)
    if with_counts:
      d_cnt(q).start(add=True)

  def drain(p, q):
    d_sca(p, q).wait()
    if with_counts:
      d_cnt(q).wait()

  load(0, 0)
  load(1, 1)
  gather(0, 0, 0)

  @pl.when(last)
  def _():
    pltpu.make_async_copy(zrows_hbm, acc.at[pl.ds(row0, ROWS_LAST)],
                          sem_z).wait()

  @pl.when(jnp.logical_not(last))
  def _():
    pltpu.make_async_copy(zrows_hbm.at[pl.ds(0, ROWS_W)],
                          acc.at[pl.ds(row0, ROWS_W)], sem_z).wait()

  plsc.subcore_barrier()

  NPER = 12

  def pipe_body(k, carry):
    base = k * NPER
    for m in range(NPER):
      i = base + m

      @pl.when(jnp.logical_and(i >= 2, i - 2 < NCHUNK))
      def _():
        drain((m - 2) % NROW, (m - 2) % NIDX)

      @pl.when(i + 2 < NCHUNK)
      def _():
        load((m + 2) % NIDX, i + 2)

      @pl.when(i + 1 < NCHUNK)
      def _():
        gather((m + 1) % NROW, (m + 1) % NIDX, i + 1)

      @pl.when(i < NCHUNK)
      def _():
        scatter(m % NROW, m % NIDX)
    return carry

  lax.fori_loop(0, (NCHUNK + 2 + NPER - 1) // NPER, pipe_body, 0)
  plsc.subcore_barrier()

  @pl.when(last)
  def _():
    pltpu.sync_copy(acc.at[pl.ds(row0, ROWS_LAST)],
                    out_hbm.at[c, pl.ds(row0, ROWS_LAST)])

  @pl.when(jnp.logical_not(last))
  def _():
    pltpu.sync_copy(acc.at[pl.ds(row0, ROWS_W)],
                    out_hbm.at[c, pl.ds(row0, ROWS_W)])

  if with_counts:
    @pl.when(s == 0)
    def _():
      pltpu.sync_copy(cnt, cntout_hbm.at[c, 0])


def _make_sc_agg(with_counts):
  mesh = plsc.VectorSubcoreMesh(core_axis_name="c", subcore_axis_name="s")
  out_type = [jax.ShapeDtypeStruct((NC, N_PAD, D), jnp.float32)]
  if with_counts:
    out_type.append(jax.ShapeDtypeStruct((NC, 1, N_CNT), jnp.float32))
  scratch_types = [
      [pltpu.VMEM((2, 1, CHUNK), jnp.int32) for _ in range(NIDX)],
      [pltpu.VMEM((CHUNK, D), jnp.float32) for _ in range(NROW)],
      pltpu.VMEM((CHUNK,), jnp.float32),
      pltpu.VMEM_SHARED((N_PAD, D), jnp.float32),
      pltpu.VMEM_SHARED((N_CNT,), jnp.float32),
      [pltpu.SemaphoreType.DMA for _ in range(NIDX)],
      [pltpu.SemaphoreType.DMA for _ in range(NROW)],
      [pltpu.SemaphoreType.DMA for _ in range(NROW)],
      [pltpu.SemaphoreType.DMA for _ in range(NIDX)],
      pltpu.SemaphoreType.DMA,
  ]

  if with_counts:
    def body(x_hbm, ei_hbm, zrows_hbm, zcnt_hbm, ones_hbm,
             out_hbm, cntout_hbm, idxb, rows, ones, acc, cnt,
             sem_r, sem_g, sem_s, sem_c, sem_z):
      _sc_agg_body(True, x_hbm, ei_hbm, zrows_hbm, zcnt_hbm,
                   ones_hbm, out_hbm, cntout_hbm, idxb, rows, ones,
                   acc, cnt, sem_r, sem_g, sem_s, sem_c, sem_z)
  else:
    def body(x_hbm, ei_hbm, zrows_hbm, zcnt_hbm, ones_hbm,
             out_hbm, idxb, rows, ones, acc, cnt,
             sem_r, sem_g, sem_s, sem_c, sem_z):
      _sc_agg_body(False, x_hbm, ei_hbm, zrows_hbm, zcnt_hbm,
                   ones_hbm, out_hbm, None, idxb, rows, ones,
                   acc, cnt, sem_r, sem_g, sem_s, sem_c, sem_z)

  return pl.kernel(body, out_type=out_type, mesh=mesh,
                   scratch_types=scratch_types)


_sc_agg_with_counts = _make_sc_agg(True)
_sc_agg_no_counts = _make_sc_agg(False)

BR = 2000


def _tc_layer_body(p_ref, c_ref, x_ref, wl_ref, wr_ref, b_ref, o_ref):
  cnt = jnp.maximum(c_ref[0] + c_ref[1], 1.0)
  agg = (p_ref[0] + p_ref[1]) / cnt
  y = lax.dot_general(agg, wl_ref[...], (((1,), (1,)), ((), ())),
                      preferred_element_type=jnp.float32)
  y = y + lax.dot_general(x_ref[...], wr_ref[...], (((1,), (1,)), ((), ())),
                          preferred_element_type=jnp.float32)
  o_ref[...] = jnp.maximum(y + b_ref[...], 0.0)


def _tc_layer(P, C, x, Wl, Wr, b):
  grid = (N_NODES // BR,)
  return pl.pallas_call(
      _tc_layer_body,
      grid=grid,
      in_specs=[
          pl.BlockSpec((NC, BR, D), lambda i: (0, i, 0)),
          pl.BlockSpec((NC, BR, 1), lambda i: (0, i, 0)),
          pl.BlockSpec((BR, D), lambda i: (i, 0)),
          pl.BlockSpec((D, D), lambda i: (0, 0)),
          pl.BlockSpec((D, D), lambda i: (0, 0)),
          pl.BlockSpec((1, D), lambda i: (0, 0)),
      ],
      out_specs=pl.BlockSpec((BR, D), lambda i: (i, 0)),
      out_shape=jax.ShapeDtypeStruct((N_NODES, D), jnp.float32),
  )(P, C.reshape(NC, N_CNT, 1), x, Wl, Wr, b.reshape(1, D))


def _tc_final_body(p_ref, c_ref, h_ref, wl_ref, wr_ref, b_ref, wc_ref,
                   bc_ref, o_ref):
  cnt = jnp.maximum(c_ref[0] + c_ref[1], 1.0)
  agg = (p_ref[0] + p_ref[1]) / cnt
  y = lax.dot_general(agg, wl_ref[...], (((1,), (1,)), ((), ())),
                      preferred_element_type=jnp.float32)
  y = y + lax.dot_general(h_ref[...], wr_ref[...], (((1,), (1,)), ((), ())),
                          preferred_element_type=jnp.float32)
  h2 = jnp.maximum(y + b_ref[...], 0.0)
  logits = jnp.sum(h2 * wc_ref[...], axis=1, keepdims=True)
  o_ref[...] = jax.nn.sigmoid(logits + bc_ref[0])


def _tc_final(P, C, h, Wl, Wr, b, Wc, bc):
  grid = (N_NODES // BR,)
  return pl.pallas_call(
      _tc_final_body,
      grid=grid,
      in_specs=[
          pl.BlockSpec((NC, BR, D), lambda i: (0, i, 0)),
          pl.BlockSpec((NC, BR, 1), lambda i: (0, i, 0)),
          pl.BlockSpec((BR, D), lambda i: (i, 0)),
          pl.BlockSpec((D, D), lambda i: (0, 0)),
          pl.BlockSpec((D, D), lambda i: (0, 0)),
          pl.BlockSpec((1, D), lambda i: (0, 0)),
          pl.BlockSpec((1, D), lambda i: (0, 0)),
          pl.BlockSpec(memory_space=pltpu.SMEM),
      ],
      out_specs=pl.BlockSpec((BR, 1), lambda i: (i, 0)),
      out_shape=jax.ShapeDtypeStruct((N_NODES, 1), jnp.float32),
  )(P, C.reshape(NC, N_CNT, 1), h, Wl, Wr, b.reshape(1, D),
    Wc.reshape(1, D), bc)


def kernel(x, edge_index, W1l, b1, W1r, W2l, b2, W2r, Wc, bc):
  n_extra = E_PAD - E_TOTAL
  pad_src = (jnp.arange(n_extra, dtype=jnp.int32) * 131) % N_NODES
  pad_dst = N_NODES + (jnp.arange(n_extra, dtype=jnp.int32) %
                       (N_PAD - N_NODES))
  ei = jnp.concatenate([edge_index, jnp.stack([pad_src, pad_dst])], axis=1)

  zrows = jnp.zeros((ROWS_LAST, D), jnp.float32)
  zcnt = jnp.zeros((N_CNT,), jnp.float32)
  ones = jnp.ones((CHUNK,), jnp.float32)

  P1, C1 = _sc_agg_with_counts(x, ei, zrows, zcnt, ones)
  h = _tc_layer(P1, C1, x, W1l, W1r, b1)
  (P2,) = _sc_agg_no_counts(h, ei, zrows, zcnt, ones)
  out = _tc_final(P2, C1, h, W2l, W2r, b2, Wc, bc)
  return out


# speedup vs baseline: 1.2252x; 1.0211x over previous
import jax
import jax.numpy as jnp
from jax import lax
from jax.experimental import pallas as pl
from jax.experimental.pallas import tpu as pltpu
from jax.experimental.pallas import tpu_sc as plsc

N_NODES = 10000
D = 128
E_TOTAL = 320000

NC = 2
NS = 16
NW = NC * NS
CHUNK = 128
E_W = 10240
NCHUNK = E_W // CHUNK
NCHUNK_TAIL = (E_TOTAL - (NW - 1) * E_W) // CHUNK
N_PAD = 10016
N_CNT = 10112
ROWS_W = 624
ROWS_LAST = N_PAD - (NS - 1) * ROWS_W
NROW = 3
NIDX = 4


def _sc_agg_body(with_counts, x_hbm, ei_hbm, zrows_hbm, zcnt_hbm,
                 ones_hbm, out_hbm, cntout_hbm, idxb, rows, ones,
                 acc, cnt, sem_r, sem_g, sem_s, sem_c, sem_z):
  c = lax.axis_index("c")
  s = lax.axis_index("s")
  wid = s * NC + c
  e0 = wid * E_W
  row0 = s * ROWS_W
  nch = jnp.where(wid == NW - 1, NCHUNK_TAIL, NCHUNK)

  last = s == NS - 1

  @pl.when(last)
  def _():
    pltpu.async_copy(zrows_hbm, acc.at[pl.ds(row0, ROWS_LAST)], sem_z)

  @pl.when(jnp.logical_not(last))
  def _():
    pltpu.async_copy(zrows_hbm.at[pl.ds(0, ROWS_W)],
                     acc.at[pl.ds(row0, ROWS_W)], sem_z)

  if with_counts:
    pltpu.sync_copy(ones_hbm, ones)

    @pl.when(s == 0)
    def _():
      pltpu.sync_copy(zcnt_hbm, cnt)

  def d_idx(q, i):
    return pltpu.make_async_copy(
        ei_hbm.at[:, pl.ds(e0 + i * CHUNK, CHUNK)], idxb[q].at[:, 0],
        sem_r[q])

  def d_gat(p, q):
    return pltpu.make_async_copy(x_hbm.at[idxb[q].at[0, 0]], rows[p],
                                 sem_g[p])

  def d_sca(p, q):
    return pltpu.make_async_copy(rows[p], acc.at[idxb[q].at[1, 0]], sem_s[p])

  def d_cnt(q):
    return pltpu.make_async_copy(ones, cnt.at[idxb[q].at[1, 0]], sem_c[q])

  def load(q, i):
    d_idx(q, i).start()

  def gather(p, q, i):
    d_idx(q, i).wait()
    d_gat(p, q).start()

  def scatter(p, q):
    d_gat(p, q).wait()
    d_sca(p, q).start(add=True)
    if with_counts:
      d_cnt(q).start(add=True)

  def drain(p, q):
    d_sca(p, q).wait()
    if with_counts:
      d_cnt(q).wait()

  load(0, 0)
  load(1, 1)
  gather(0, 0, 0)

  @pl.when(last)
  def _():
    pltpu.make_async_copy(zrows_hbm, acc.at[pl.ds(row0, ROWS_LAST)],
                          sem_z).wait()

  @pl.when(jnp.logical_not(last))
  def _():
    pltpu.make_async_copy(zrows_hbm.at[pl.ds(0, ROWS_W)],
                          acc.at[pl.ds(row0, ROWS_W)], sem_z).wait()

  plsc.subcore_barrier()

  NPER = 12

  def pipe_body(k, carry):
    base = k * NPER
    for m in range(NPER):
      i = base + m

      @pl.when(jnp.logical_and(i >= 2, i - 2 < nch))
      def _():
        drain((m - 2) % NROW, (m - 2) % NIDX)

      @pl.when(i + 2 < nch)
      def _():
        load((m + 2) % NIDX, i + 2)

      @pl.when(i + 1 < nch)
      def _():
        gather((m + 1) % NROW, (m + 1) % NIDX, i + 1)

      @pl.when(i < nch)
      def _():
        scatter(m % NROW, m % NIDX)
    return carry

  lax.fori_loop(0, (NCHUNK + 2 + NPER - 1) // NPER, pipe_body, 0)
  plsc.subcore_barrier()

  @pl.when(last)
  def _():
    pltpu.sync_copy(acc.at[pl.ds(row0, ROWS_LAST)],
                    out_hbm.at[c, pl.ds(row0, ROWS_LAST)])

  @pl.when(jnp.logical_not(last))
  def _():
    pltpu.sync_copy(acc.at[pl.ds(row0, ROWS_W)],
                    out_hbm.at[c, pl.ds(row0, ROWS_W)])

  if with_counts:
    @pl.when(s == 0)
    def _():
      pltpu.sync_copy(cnt, cntout_hbm.at[c, 0])


def _make_sc_agg(with_counts):
  mesh = plsc.VectorSubcoreMesh(core_axis_name="c", subcore_axis_name="s")
  out_type = [jax.ShapeDtypeStruct((NC, N_PAD, D), jnp.float32)]
  if with_counts:
    out_type.append(jax.ShapeDtypeStruct((NC, 1, N_CNT), jnp.float32))
  scratch_types = [
      [pltpu.VMEM((2, 1, CHUNK), jnp.int32) for _ in range(NIDX)],
      [pltpu.VMEM((CHUNK, D), jnp.float32) for _ in range(NROW)],
      pltpu.VMEM((CHUNK,), jnp.float32),
      pltpu.VMEM_SHARED((N_PAD, D), jnp.float32),
      pltpu.VMEM_SHARED((N_CNT,), jnp.float32),
      [pltpu.SemaphoreType.DMA for _ in range(NIDX)],
      [pltpu.SemaphoreType.DMA for _ in range(NROW)],
      [pltpu.SemaphoreType.DMA for _ in range(NROW)],
      [pltpu.SemaphoreType.DMA for _ in range(NIDX)],
      pltpu.SemaphoreType.DMA,
  ]

  if with_counts:
    def body(x_hbm, ei_hbm, zrows_hbm, zcnt_hbm, ones_hbm,
             out_hbm, cntout_hbm, idxb, rows, ones, acc, cnt,
             sem_r, sem_g, sem_s, sem_c, sem_z):
      _sc_agg_body(True, x_hbm, ei_hbm, zrows_hbm, zcnt_hbm,
                   ones_hbm, out_hbm, cntout_hbm, idxb, rows, ones,
                   acc, cnt, sem_r, sem_g, sem_s, sem_c, sem_z)
  else:
    def body(x_hbm, ei_hbm, zrows_hbm, zcnt_hbm, ones_hbm,
             out_hbm, idxb, rows, ones, acc, cnt,
             sem_r, sem_g, sem_s, sem_c, sem_z):
      _sc_agg_body(False, x_hbm, ei_hbm, zrows_hbm, zcnt_hbm,
                   ones_hbm, out_hbm, None, idxb, rows, ones,
                   acc, cnt, sem_r, sem_g, sem_s, sem_c, sem_z)

  return pl.kernel(body, out_type=out_type, mesh=mesh,
                   scratch_types=scratch_types)


_sc_agg_with_counts = _make_sc_agg(True)
_sc_agg_no_counts = _make_sc_agg(False)

BR = 2000


def _tc_layer_body(p_ref, c_ref, x_ref, wl_ref, wr_ref, b_ref, o_ref):
  cnt = jnp.maximum(c_ref[0] + c_ref[1], 1.0)
  agg = (p_ref[0] + p_ref[1]) / cnt
  y = lax.dot_general(agg, wl_ref[...], (((1,), (1,)), ((), ())),
                      preferred_element_type=jnp.float32)
  y = y + lax.dot_general(x_ref[...], wr_ref[...], (((1,), (1,)), ((), ())),
                          preferred_element_type=jnp.float32)
  o_ref[...] = jnp.maximum(y + b_ref[...], 0.0)


def _tc_layer(P, C, x, Wl, Wr, b):
  grid = (N_NODES // BR,)
  return pl.pallas_call(
      _tc_layer_body,
      grid=grid,
      in_specs=[
          pl.BlockSpec((NC, BR, D), lambda i: (0, i, 0)),
          pl.BlockSpec((NC, BR, 1), lambda i: (0, i, 0)),
          pl.BlockSpec((BR, D), lambda i: (i, 0)),
          pl.BlockSpec((D, D), lambda i: (0, 0)),
          pl.BlockSpec((D, D), lambda i: (0, 0)),
          pl.BlockSpec((1, D), lambda i: (0, 0)),
      ],
      out_specs=pl.BlockSpec((BR, D), lambda i: (i, 0)),
      out_shape=jax.ShapeDtypeStruct((N_NODES, D), jnp.float32),
  )(P, C.reshape(NC, N_CNT, 1), x, Wl, Wr, b.reshape(1, D))


def _tc_final_body(p_ref, c_ref, h_ref, wl_ref, wr_ref, b_ref, wc_ref,
                   bc_ref, o_ref):
  cnt = jnp.maximum(c_ref[0] + c_ref[1], 1.0)
  agg = (p_ref[0] + p_ref[1]) / cnt
  y = lax.dot_general(agg, wl_ref[...], (((1,), (1,)), ((), ())),
                      preferred_element_type=jnp.float32)
  y = y + lax.dot_general(h_ref[...], wr_ref[...], (((1,), (1,)), ((), ())),
                          preferred_element_type=jnp.float32)
  h2 = jnp.maximum(y + b_ref[...], 0.0)
  logits = jnp.sum(h2 * wc_ref[...], axis=1, keepdims=True)
  o_ref[...] = jax.nn.sigmoid(logits + bc_ref[0])


def _tc_final(P, C, h, Wl, Wr, b, Wc, bc):
  grid = (N_NODES // BR,)
  return pl.pallas_call(
      _tc_final_body,
      grid=grid,
      in_specs=[
          pl.BlockSpec((NC, BR, D), lambda i: (0, i, 0)),
          pl.BlockSpec((NC, BR, 1), lambda i: (0, i, 0)),
          pl.BlockSpec((BR, D), lambda i: (i, 0)),
          pl.BlockSpec((D, D), lambda i: (0, 0)),
          pl.BlockSpec((D, D), lambda i: (0, 0)),
          pl.BlockSpec((1, D), lambda i: (0, 0)),
          pl.BlockSpec((1, D), lambda i: (0, 0)),
          pl.BlockSpec(memory_space=pltpu.SMEM),
      ],
      out_specs=pl.BlockSpec((BR, 1), lambda i: (i, 0)),
      out_shape=jax.ShapeDtypeStruct((N_NODES, 1), jnp.float32),
  )(P, C.reshape(NC, N_CNT, 1), h, Wl, Wr, b.reshape(1, D),
    Wc.reshape(1, D), bc)


def kernel(x, edge_index, W1l, b1, W1r, W2l, b2, W2r, Wc, bc):
  ei = edge_index
  zrows = jnp.zeros((ROWS_LAST, D), jnp.float32)
  zcnt = jnp.zeros((N_CNT,), jnp.float32)
  ones = jnp.ones((CHUNK,), jnp.float32)

  P1, C1 = _sc_agg_with_counts(x, ei, zrows, zcnt, ones)
  h = _tc_layer(P1, C1, x, W1l, W1r, b1)
  (P2,) = _sc_agg_no_counts(h, ei, zrows, zcnt, ones)
  out = _tc_final(P2, C1, h, W2l, W2r, b2, Wc, bc)
  return out
